# baseline (speedup 1.0000x reference)
import functools

import jax
import jax.numpy as jnp
from jax import lax
from jax.experimental import pallas as pl
from jax.experimental.pallas import tpu as pltpu
from jax.experimental.pallas import tpu_sc as plsc

N = 10000
E = 320000
D = 128
H = 128
G = 128
NOUT = 2

NC = 2
NS = 16
NW = NC * NS
C = 96
K = (E // NW + C - 1) // C
EW = K * C
EPAD = EW - E // NW

ACC_ROWS = 10240
RPT = ACC_ROWS // NS

NDEG = 10240

R = 512
NBLK = 20


def _sc_mesh():
    return plsc.VectorSubcoreMesh(core_axis_name="c", subcore_axis_name="s")


def _deg_body(dstp_hbm, degp_hbm, dstv, degv):
    cid = lax.axis_index("c")
    sid = lax.axis_index("s")
    w = cid * NS + sid
    pltpu.sync_copy(dstp_hbm.at[pl.ds(w * EW, EW)], dstv)

    @pl.loop(0, NDEG // 16)
    def _zero(i):
        degv[pl.ds(i * 16, 16)] = jnp.zeros((16,), jnp.float32)

    ones = jnp.ones((16,), jnp.float32)

    @pl.loop(0, EW // 16)
    def _count(i):
        idx = dstv[pl.ds(i * 16, 16)]
        plsc.addupdate_scatter(degv, [idx], ones)

    pltpu.sync_copy(degv, degp_hbm.at[w])


def _deg_call(dstp):
    fn = pl.kernel(
        _deg_body,
        out_type=jax.ShapeDtypeStruct((NW, NDEG), jnp.float32),
        mesh=_sc_mesh(),
        compiler_params=pltpu.CompilerParams(needs_layout_passes=False),
        scratch_types=[
            pltpu.VMEM((EW,), jnp.int32),
            pltpu.VMEM((NDEG,), jnp.float32),
        ],
    )
    return fn(dstp)


def _scat_body(m_hbm, srcp_hbm, dstp3_hbm, spart_hbm, srcv, dstv, rows, acc,
               gsem):
    cid = lax.axis_index("c")
    sid = lax.axis_index("s")
    w = cid * NS + sid

    def gather(j, r):
        pltpu.async_copy(m_hbm.at[srcv.at[pl.ds(j * C, C)]], rows.at[r],
                         gsem.at[r])

    def wait_gather(r):
        pltpu.make_async_copy(m_hbm.at[srcv.at[pl.ds(0, C)]], rows.at[r],
                              gsem.at[r]).wait()

    def scatter(j, r):
        pltpu.sync_copy(rows.at[r], acc.at[dstv.at[j]], add=True)

    for r in range(16):
        for q in range(H // 16):
            rows[0, r, pl.ds(q * 16, 16)] = jnp.zeros((16,), jnp.float32)

    pltpu.sync_copy(srcp_hbm.at[pl.ds(w * EW, EW)], srcv)
    gather(0, 1)
    pltpu.sync_copy(dstp3_hbm.at[w], dstv)

    zbuf = rows.at[0].at[pl.ds(0, 16)]
    base = sid * RPT

    @pl.loop(0, RPT // 16)
    def _zacc(k):
        pltpu.sync_copy(zbuf, acc.at[pl.ds(base + k * 16, 16)])

    plsc.subcore_barrier()

    @pl.loop(0, (K - 1) // 2)
    def _pairs(t):
        g = t * 2
        gather(g + 1, 0)
        wait_gather(1)
        scatter(g, 1)
        gather(g + 2, 1)
        wait_gather(0)
        scatter(g + 1, 0)

    wait_gather(1)
    scatter(K - 1, 1)

    plsc.subcore_barrier()

    pltpu.sync_copy(acc.at[pl.ds(base, RPT)], spart_hbm.at[cid].at[pl.ds(base, RPT)])


def _scat_call(m, srcp, dstp3):
    fn = pl.kernel(
        _scat_body,
        out_type=jax.ShapeDtypeStruct((NC, ACC_ROWS, H), jnp.float32),
        mesh=_sc_mesh(),
        scratch_types=[
            pltpu.VMEM((EW,), jnp.int32),
            pltpu.VMEM((K, C), jnp.int32),
            pltpu.VMEM((2, C, H), jnp.float32),
            pltpu.VMEM_SHARED((ACC_ROWS, H), jnp.float32),
            pltpu.SemaphoreType.DMA((2,)),
        ],
    )
    return fn(m, srcp, dstp3)


def _prep_body(degp_ref, x_ref, we_ref, be_ref, cw_ref, m_ref, dinv_ref):
    deg = jnp.sum(degp_ref[...], axis=0) + 1.0
    dv = lax.rsqrt(deg)[:, None]
    h = jnp.maximum(
        jnp.dot(x_ref[...], we_ref[...], preferred_element_type=jnp.float32)
        + be_ref[...][None, :], 0.0)
    m_ref[...] = jnp.dot(h, cw_ref[...], preferred_element_type=jnp.float32) * dv
    dinv_ref[...] = dv


def _prep_call(degp, x, we, be, cw0):
    return pl.pallas_call(
        _prep_body,
        grid=(NBLK,),
        in_specs=[
            pl.BlockSpec((NW, R), lambda i: (0, i)),
            pl.BlockSpec((R, D), lambda i: (i, 0)),
            pl.BlockSpec((D, H), lambda i: (0, 0)),
            pl.BlockSpec((H,), lambda i: (0,)),
            pl.BlockSpec((H, H), lambda i: (0, 0)),
        ],
        out_specs=[
            pl.BlockSpec((R, H), lambda i: (i, 0)),
            pl.BlockSpec((R, 1), lambda i: (i, 0)),
        ],
        out_shape=[
            jax.ShapeDtypeStruct((N, H), jnp.float32),
            jax.ShapeDtypeStruct((N, 1), jnp.float32),
        ],
    )(degp, x, we, be, cw0)


def _mid_body(spart_ref, m_ref, dinv_ref, cb_ref, cw_ref, mout_ref):
    sp = spart_ref[...]
    s = sp[0] + sp[1] + m_ref[...]
    h = jnp.maximum(s * dinv_ref[...] + cb_ref[...][None, :], 0.0)
    mout_ref[...] = jnp.dot(h, cw_ref[...], preferred_element_type=jnp.float32) * dinv_ref[...]


def _mid_call(spart, m, dinv, cb, cwn):
    return pl.pallas_call(
        _mid_body,
        grid=(NBLK,),
        in_specs=[
            pl.BlockSpec((NC, R, H), lambda i: (0, i, 0)),
            pl.BlockSpec((R, H), lambda i: (i, 0)),
            pl.BlockSpec((R, 1), lambda i: (i, 0)),
            pl.BlockSpec((H,), lambda i: (0,)),
            pl.BlockSpec((H, H), lambda i: (0, 0)),
        ],
        out_specs=pl.BlockSpec((R, H), lambda i: (i, 0)),
        out_shape=jax.ShapeDtypeStruct((N, H), jnp.float32),
    )(spart, m, dinv, cb, cwn)


def _fin_body(spart_ref, m_ref, dinv_ref, cb_ref, batch_ref,
              fw0_ref, fb0_ref, fw1_ref, fb1_ref, fw2_ref, fb2_ref,
              ow_ref, ob_ref, out_ref, gacc):
    i = pl.program_id(0)
    sp = spart_ref[...]
    s = sp[0] + sp[1] + m_ref[...]
    h = jnp.maximum(s * dinv_ref[...] + cb_ref[...][None, :], 0.0)

    rows_c = i * R + lax.broadcasted_iota(jnp.int32, (R, 1), 0)
    h = jnp.where(rows_c < N, h, 0.0)

    rows_r = i * R + lax.broadcasted_iota(jnp.int32, (1, R), 1)
    gid = lax.broadcasted_iota(jnp.int32, (G, 1), 0)
    onehot = ((gid == batch_ref[...][None, :]) & (rows_r < N)).astype(jnp.float32)

    @pl.when(i == 0)
    def _():
        gacc[...] = jnp.zeros_like(gacc)

    gacc[...] += jnp.dot(onehot, h, preferred_element_type=jnp.float32)

    @pl.when(i == NBLK - 1)
    def _():
        g = gacc[...]
        for fw_ref, fb_ref in ((fw0_ref, fb0_ref), (fw1_ref, fb1_ref), (fw2_ref, fb2_ref)):
            g = jnp.maximum(
                jnp.dot(g, fw_ref[...], preferred_element_type=jnp.float32)
                + fb_ref[...][None, :], 0.0)
        lo = jnp.dot(g, ow_ref[...], preferred_element_type=jnp.float32) + ob_ref[...][None, :]
        sh = lo - jnp.max(lo, axis=1, keepdims=True)
        out_ref[...] = sh - jnp.log(jnp.sum(jnp.exp(sh), axis=1, keepdims=True))


def _fin_call(spart, m, dinv, cb2, batch, fw0, fb0, fw1, fb1, fw2, fb2, ow, ob):
    return pl.pallas_call(
        _fin_body,
        grid=(NBLK,),
        in_specs=[
            pl.BlockSpec((NC, R, H), lambda i: (0, i, 0)),
            pl.BlockSpec((R, H), lambda i: (i, 0)),
            pl.BlockSpec((R, 1), lambda i: (i, 0)),
            pl.BlockSpec((H,), lambda i: (0,)),
            pl.BlockSpec((R,), lambda i: (i,)),
            pl.BlockSpec((H, H), lambda i: (0, 0)),
            pl.BlockSpec((H,), lambda i: (0,)),
            pl.BlockSpec((H, H), lambda i: (0, 0)),
            pl.BlockSpec((H,), lambda i: (0,)),
            pl.BlockSpec((H, H), lambda i: (0, 0)),
            pl.BlockSpec((H,), lambda i: (0,)),
            pl.BlockSpec((H, NOUT), lambda i: (0, 0)),
            pl.BlockSpec((NOUT,), lambda i: (0,)),
        ],
        out_specs=pl.BlockSpec((G, NOUT), lambda i: (0, 0)),
        out_shape=jax.ShapeDtypeStruct((G, NOUT), jnp.float32),
        scratch_shapes=[pltpu.VMEM((G, H), jnp.float32)],
    )(spart, m, dinv, cb2, batch, fw0, fb0, fw1, fb1, fw2, fb2, ow, ob)


def kernel(x, edge_index, batch, we, be, cw0, cb0, cw1, cb1, cw2, cb2,
           fw0, fb0, fw1, fb1, fw2, fb2, ow, ob):
    src = edge_index[0]
    dst = edge_index[1]
    srcp = jnp.concatenate(
        [src.reshape(NW, E // NW),
         jnp.zeros((NW, EPAD), jnp.int32)], axis=1).reshape(-1)
    pad_dst = (N + (jnp.arange(NW, dtype=jnp.int32)[:, None] * EPAD
                    + jnp.arange(EPAD, dtype=jnp.int32)[None, :])
               % (ACC_ROWS - N)).astype(jnp.int32)
    dstp2 = jnp.concatenate([dst.reshape(NW, E // NW), pad_dst], axis=1)
    dstp3 = dstp2.reshape(NW, K, C)
    dstp = dstp2.reshape(-1)

    degp = _deg_call(dstp)
    m, dinv = _prep_call(degp, x, we, be, cw0)
    for cb, cwn in ((cb0, cw1), (cb1, cw2)):
        spart = _scat_call(m, srcp, dstp3)
        m = _mid_call(spart, m, dinv, cb, cwn)
    spart = _scat_call(m, srcp, dstp3)
    return _fin_call(spart, m, dinv, cb2, batch,
                     fw0, fb0, fw1, fb1, fw2, fb2, ow, ob)


# speedup vs baseline: 1.5599x; 1.5599x over previous
import jax
import jax.numpy as jnp
from jax import lax
from jax.experimental import pallas as pl
from jax.experimental.pallas import tpu as pltpu
from jax.experimental.pallas import tpu_sc as plsc

N = 10000
E = 320000
D = 128
H = 128
G = 128
NOUT = 2

NC = 2
NS = 16
NW = NC * NS
C = 80
K = E // NW // C
EW = K * C

ACC_ROWS = 10240
RPT = ACC_ROWS // NS

NDEG = 10240

R = 512
NBLK = 20


def _sc_mesh():
    return plsc.VectorSubcoreMesh(core_axis_name="c", subcore_axis_name="s")


def _deg_body(dstp_hbm, degp_hbm, dstv, degv):
    cid = lax.axis_index("c")
    sid = lax.axis_index("s")
    w = cid * NS + sid
    pltpu.sync_copy(dstp_hbm.at[pl.ds(w * EW, EW)], dstv)

    @pl.loop(0, NDEG // 16)
    def _zero(i):
        degv[pl.ds(i * 16, 16)] = jnp.zeros((16,), jnp.float32)

    ones = jnp.ones((16,), jnp.float32)

    @pl.loop(0, EW // 16)
    def _count(i):
        idx = dstv[pl.ds(i * 16, 16)]
        plsc.addupdate_scatter(degv, [idx], ones)

    pltpu.sync_copy(degv, degp_hbm.at[w])


def _deg_call(dstp):
    fn = pl.kernel(
        _deg_body,
        out_type=jax.ShapeDtypeStruct((NW, NDEG), jnp.float32),
        mesh=_sc_mesh(),
        compiler_params=pltpu.CompilerParams(needs_layout_passes=False),
        scratch_types=[
            pltpu.VMEM((EW,), jnp.int32),
            pltpu.VMEM((NDEG,), jnp.float32),
        ],
    )
    return fn(dstp)


def _scat_body(m_hbm, srcp_hbm, dstp3_hbm, spart_hbm, srcv, dstv, rows, acc,
               gsem):
    cid = lax.axis_index("c")
    sid = lax.axis_index("s")
    w = cid * NS + sid

    def gather(j, r):
        pltpu.async_copy(m_hbm.at[srcv.at[pl.ds(j * C, C)]], rows.at[r],
                         gsem.at[r])

    def wait_gather(r):
        pltpu.make_async_copy(m_hbm.at[srcv.at[pl.ds(0, C)]], rows.at[r],
                              gsem.at[r]).wait()

    def scatter(j, r):
        pltpu.sync_copy(rows.at[r], acc.at[dstv.at[j]], add=True)

    for r in range(16):
        for q in range(H // 16):
            rows[0, r, pl.ds(q * 16, 16)] = jnp.zeros((16,), jnp.float32)

    pltpu.sync_copy(srcp_hbm.at[pl.ds(w * EW, EW)], srcv)
    gather(0, 1)
    pltpu.sync_copy(dstp3_hbm.at[w], dstv)

    zbuf = rows.at[0].at[pl.ds(0, 16)]
    base = sid * RPT

    @pl.loop(0, RPT // 16)
    def _zacc(k):
        pltpu.sync_copy(zbuf, acc.at[pl.ds(base + k * 16, 16)])

    plsc.subcore_barrier()

    @pl.loop(0, (K - 1) // 2)
    def _pairs(t):
        g = t * 2
        gather(g + 1, 0)
        wait_gather(1)
        scatter(g, 1)
        gather(g + 2, 1)
        wait_gather(0)
        scatter(g + 1, 0)

    wait_gather(1)
    scatter(K - 1, 1)

    plsc.subcore_barrier()

    pltpu.sync_copy(acc.at[pl.ds(base, RPT)], spart_hbm.at[cid].at[pl.ds(base, RPT)])


def _scat_call(m, srcp, dstp3):
    fn = pl.kernel(
        _scat_body,
        out_type=jax.ShapeDtypeStruct((NC, ACC_ROWS, H), jnp.float32),
        mesh=_sc_mesh(),
        scratch_types=[
            pltpu.VMEM((EW,), jnp.int32),
            pltpu.VMEM((K, C), jnp.int32),
            pltpu.VMEM((2, C, H), jnp.float32),
            pltpu.VMEM_SHARED((ACC_ROWS, H), jnp.float32),
            pltpu.SemaphoreType.DMA((2,)),
        ],
    )
    return fn(m, srcp, dstp3)


def _prep_body(degp_ref, x_ref, we_ref, be_ref, cw_ref, m_ref, dinv_ref):
    deg = jnp.sum(degp_ref[...], axis=0) + 1.0
    dv = lax.rsqrt(deg)[:, None]
    h = jnp.maximum(
        jnp.dot(x_ref[...], we_ref[...], preferred_element_type=jnp.float32)
        + be_ref[...][None, :], 0.0)
    m_ref[...] = jnp.dot(h, cw_ref[...], preferred_element_type=jnp.float32) * dv
    dinv_ref[...] = dv


def _prep_call(degp, x, we, be, cw0):
    return pl.pallas_call(
        _prep_body,
        grid=(NBLK,),
        in_specs=[
            pl.BlockSpec((NW, R), lambda i: (0, i)),
            pl.BlockSpec((R, D), lambda i: (i, 0)),
            pl.BlockSpec((D, H), lambda i: (0, 0)),
            pl.BlockSpec((H,), lambda i: (0,)),
            pl.BlockSpec((H, H), lambda i: (0, 0)),
        ],
        out_specs=[
            pl.BlockSpec((R, H), lambda i: (i, 0)),
            pl.BlockSpec((R, 1), lambda i: (i, 0)),
        ],
        out_shape=[
            jax.ShapeDtypeStruct((N, H), jnp.float32),
            jax.ShapeDtypeStruct((N, 1), jnp.float32),
        ],
    )(degp, x, we, be, cw0)


def _mid_body(spart_ref, m_ref, dinv_ref, cb_ref, cw_ref, mout_ref):
    sp = spart_ref[...]
    s = sp[0] + sp[1] + m_ref[...]
    h = jnp.maximum(s * dinv_ref[...] + cb_ref[...][None, :], 0.0)
    mout_ref[...] = jnp.dot(h, cw_ref[...], preferred_element_type=jnp.float32) * dinv_ref[...]


def _mid_call(spart, m, dinv, cb, cwn):
    return pl.pallas_call(
        _mid_body,
        grid=(NBLK,),
        in_specs=[
            pl.BlockSpec((NC, R, H), lambda i: (0, i, 0)),
            pl.BlockSpec((R, H), lambda i: (i, 0)),
            pl.BlockSpec((R, 1), lambda i: (i, 0)),
            pl.BlockSpec((H,), lambda i: (0,)),
            pl.BlockSpec((H, H), lambda i: (0, 0)),
        ],
        out_specs=pl.BlockSpec((R, H), lambda i: (i, 0)),
        out_shape=jax.ShapeDtypeStruct((N, H), jnp.float32),
    )(spart, m, dinv, cb, cwn)


def _fin_body(spart_ref, m_ref, dinv_ref, cb_ref, batch_ref,
              fw0_ref, fb0_ref, fw1_ref, fb1_ref, fw2_ref, fb2_ref,
              ow_ref, ob_ref, out_ref, gacc):
    i = pl.program_id(0)
    sp = spart_ref[...]
    s = sp[0] + sp[1] + m_ref[...]
    h = jnp.maximum(s * dinv_ref[...] + cb_ref[...][None, :], 0.0)

    rows_c = i * R + lax.broadcasted_iota(jnp.int32, (R, 1), 0)
    h = jnp.where(rows_c < N, h, 0.0)

    rows_r = i * R + lax.broadcasted_iota(jnp.int32, (1, R), 1)
    gid = lax.broadcasted_iota(jnp.int32, (G, 1), 0)
    onehot = ((gid == batch_ref[...][None, :]) & (rows_r < N)).astype(jnp.float32)

    @pl.when(i == 0)
    def _():
        gacc[...] = jnp.zeros_like(gacc)

    gacc[...] += jnp.dot(onehot, h, preferred_element_type=jnp.float32)

    @pl.when(i == NBLK - 1)
    def _():
        g = gacc[...]
        for fw_ref, fb_ref in ((fw0_ref, fb0_ref), (fw1_ref, fb1_ref), (fw2_ref, fb2_ref)):
            g = jnp.maximum(
                jnp.dot(g, fw_ref[...], preferred_element_type=jnp.float32)
                + fb_ref[...][None, :], 0.0)
        lo = jnp.dot(g, ow_ref[...], preferred_element_type=jnp.float32) + ob_ref[...][None, :]
        sh = lo - jnp.max(lo, axis=1, keepdims=True)
        out_ref[...] = sh - jnp.log(jnp.sum(jnp.exp(sh), axis=1, keepdims=True))


def _fin_call(spart, m, dinv, cb2, batch, fw0, fb0, fw1, fb1, fw2, fb2, ow, ob):
    return pl.pallas_call(
        _fin_body,
        grid=(NBLK,),
        in_specs=[
            pl.BlockSpec((NC, R, H), lambda i: (0, i, 0)),
            pl.BlockSpec((R, H), lambda i: (i, 0)),
            pl.BlockSpec((R, 1), lambda i: (i, 0)),
            pl.BlockSpec((H,), lambda i: (0,)),
            pl.BlockSpec((R,), lambda i: (i,)),
            pl.BlockSpec((H, H), lambda i: (0, 0)),
            pl.BlockSpec((H,), lambda i: (0,)),
            pl.BlockSpec((H, H), lambda i: (0, 0)),
            pl.BlockSpec((H,), lambda i: (0,)),
            pl.BlockSpec((H, H), lambda i: (0, 0)),
            pl.BlockSpec((H,), lambda i: (0,)),
            pl.BlockSpec((H, NOUT), lambda i: (0, 0)),
            pl.BlockSpec((NOUT,), lambda i: (0,)),
        ],
        out_specs=pl.BlockSpec((G, NOUT), lambda i: (0, 0)),
        out_shape=jax.ShapeDtypeStruct((G, NOUT), jnp.float32),
        scratch_shapes=[pltpu.VMEM((G, H), jnp.float32)],
    )(spart, m, dinv, cb2, batch, fw0, fb0, fw1, fb1, fw2, fb2, ow, ob)


def kernel(x, edge_index, batch, we, be, cw0, cb0, cw1, cb1, cw2, cb2,
           fw0, fb0, fw1, fb1, fw2, fb2, ow, ob):
    src = edge_index[0]
    dst = edge_index[1]
    srcp = src
    dstp = dst
    dstp3 = dst.reshape(NW, K, C)

    degp = _deg_call(dstp)
    m, dinv = _prep_call(degp, x, we, be, cw0)
    for cb, cwn in ((cb0, cw1), (cb1, cw2)):
        spart = _scat_call(m, srcp, dstp3)
        m = _mid_call(spart, m, dinv, cb, cwn)
    spart = _scat_call(m, srcp, dstp3)
    return _fin_call(spart, m, dinv, cb2, batch,
                     fw0, fb0, fw1, fb1, fw2, fb2, ow, ob)
